# baseline (speedup 1.0000x reference)
import functools

import jax
import jax.numpy as jnp
from jax import lax
from jax.experimental import pallas as pl
from jax.experimental.pallas import tpu as pltpu
from jax.experimental.pallas import tpu_sc as plsc

_MAX_OBJ = 100
_IOU_THRES = 0.45
_SCORE_THRES = 0.25
_MAX_WH = 640.0
_NEG_INF = float("-inf")
_BIG_I32 = 2**30
_ROWL = 16


def _preproc_kernel(x_ref, scr_o, nx1_o, ny1_o, nx2_o, ny2_o, a2_o, *, NCLS):
    cx = x_ref[0, 0]
    cy = x_ref[0, 1]
    w = x_ref[0, 2]
    h = x_ref[0, 3]
    conf = x_ref[0, 4]

    cxr = cx.astype(jnp.bfloat16).astype(jnp.float32)
    cyr = cy.astype(jnp.bfloat16).astype(jnp.float32)
    wr = w.astype(jnp.bfloat16).astype(jnp.float32)
    hr = h.astype(jnp.bfloat16).astype(jnp.float32)
    bx1 = cxr - 0.5 * wr
    by1 = cyr - 0.5 * hr
    bx2 = cxr + 0.5 * wr
    by2 = cyr + 0.5 * hr

    sc0 = x_ref[0, 5] * conf

    def cls_body(c, carry):
        msc, cat = carry
        sc = x_ref[0, 5 + c] * conf
        cat = jnp.where(sc > msc, c, cat)
        msc = jnp.maximum(msc, sc)
        return msc, cat

    msc, cat = lax.fori_loop(1, NCLS, cls_body,
                             (sc0, jnp.zeros_like(sc0, jnp.int32)))
    catf = cat.astype(jnp.float32)

    nx1 = bx1 + catf * _MAX_WH
    ny1 = by1 + catf * _MAX_WH
    nx2 = bx2 + catf * _MAX_WH
    ny2 = by2 + catf * _MAX_WH

    scr_o[0] = jnp.where(msc > _SCORE_THRES, msc, _NEG_INF)
    nx1_o[0] = nx1
    ny1_o[0] = ny1
    nx2_o[0] = nx2
    ny2_o[0] = ny2
    a2_o[0] = (nx2 - nx1) * (ny2 - ny1)


def _xlane_reduce(v, op):
    lanes = lax.iota(jnp.int32, 16)
    for k in (1, 2, 4, 8):
        v = op(v, v.at[lanes ^ k].get(mode="promise_in_bounds"))
    return v


def _sc_nms_kernel(scr_h, nx1_h, ny1_h, nx2_h, ny2_h, a2_h, out_h,
                   scr_v, x1_v, y1_v, x2_v, y2_v, a2_v, rows_v,
                   *, B, NP):
    wid = lax.axis_index("s") * 2 + lax.axis_index("c")
    CH = NP // 16
    lanes = lax.iota(jnp.int32, 16)

    @pl.when(wid < B)
    def _run():
        b = wid
        bf = b.astype(jnp.float32)
        pltpu.sync_copy(scr_h.at[b], scr_v)
        pltpu.sync_copy(nx1_h.at[b], x1_v)
        pltpu.sync_copy(ny1_h.at[b], y1_v)
        pltpu.sync_copy(nx2_h.at[b], x2_v)
        pltpu.sync_copy(ny2_h.at[b], y2_v)
        pltpu.sync_copy(a2_h.at[b], a2_v)

        neg = jnp.full((16,), _NEG_INF, jnp.float32)
        zero_i = jnp.zeros((16,), jnp.int32)

        def scan0_body(c, carry):
            vm, vi = carry
            s = scr_v[pl.ds(c * 16, 16)]
            upd = s > vm
            return jnp.maximum(s, vm), jnp.where(upd, c, vi)

        vmax, vidxc = lax.fori_loop(0, CH, scan0_body, (neg, zero_i),
                                    unroll=4)

        def pick_body(i, carry):
            vm, vi = carry
            m = _xlane_reduce(vm, jnp.maximum)[0]
            ok = m > _NEG_INF
            cand = jnp.where(vm == m, vi * 16 + lanes, _BIG_I32)
            j = _xlane_reduce(cand, jnp.minimum)[0]

            cj16 = (j // 16) * 16
            jl = jnp.broadcast_to(j & 15, (16,))

            def splat_at(arr_ref):
                v = arr_ref[pl.ds(cj16, 16)]
                return v.at[jl].get(mode="promise_in_bounds")

            x1j = splat_at(x1_v)
            y1j = splat_at(y1_v)
            x2j = splat_at(x2_v)
            y2j = splat_at(y2_v)
            a1 = (x2j - x1j) * (y2j - y1j)

            catf = ((x1j + 0.5) / _MAX_WH).astype(jnp.int32) \
                .astype(jnp.float32)
            off = catf * _MAX_WH
            scorej = jnp.where(ok, m, 0.0)
            okf = jnp.where(ok, 1.0, 0.0)
            row = jnp.where(lanes == 0, bf,
                  jnp.where(lanes == 1, x1j - off,
                  jnp.where(lanes == 2, y1j - off,
                  jnp.where(lanes == 3, x2j - off,
                  jnp.where(lanes == 4, y2j - off,
                  jnp.where(lanes == 5, catf,
                  jnp.where(lanes == 6, scorej, 0.0))))))) * okf
            rows_v[pl.ds(i * _ROWL, 16)] = row

            sj = scr_v[pl.ds(cj16, 16)]
            scr_v[pl.ds(cj16, 16)] = jnp.where(lanes == (j & 15),
                                               _NEG_INF, sj)

            def chunk_body(c, carry2):
                vm2, vi2 = carry2
                ds = pl.ds(c * 16, 16)
                s = scr_v[ds]
                xx1 = jnp.maximum(x1j, x1_v[ds])
                yy1 = jnp.maximum(y1j, y1_v[ds])
                xx2 = jnp.minimum(x2j, x2_v[ds])
                yy2 = jnp.minimum(y2j, y2_v[ds])
                inter = (jnp.maximum(xx2 - xx1, 0.0)
                         * jnp.maximum(yy2 - yy1, 0.0))
                iou = inter / (a1 + a2_v[ds] - inter + 1e-9)
                news = jnp.where(iou > _IOU_THRES, neg, s)
                scr_v[ds] = news
                upd = news > vm2
                return jnp.maximum(news, vm2), jnp.where(upd, c, vi2)

            return lax.fori_loop(0, CH, chunk_body, (neg, zero_i),
                                 unroll=2)

        lax.fori_loop(0, _MAX_OBJ, pick_body, (vmax, vidxc))
        pltpu.sync_copy(rows_v, out_h.at[b])


@jax.jit
def kernel(x):
    B, N, C = x.shape
    R = (N + 127) // 128
    NP = R * 128
    NCLS = C - 5

    xp = jnp.pad(x, ((0, 0), (0, NP - N), (0, 0)))
    xt = xp.transpose(0, 2, 1).reshape(B, C, R, 128)

    slab = jax.ShapeDtypeStruct((B, R, 128), jnp.float32)
    ospec = pl.BlockSpec((1, R, 128), lambda b: (b, 0, 0))
    slabs = pl.pallas_call(
        functools.partial(_preproc_kernel, NCLS=NCLS),
        grid=(B,),
        in_specs=[pl.BlockSpec((1, C, R, 128), lambda b: (b, 0, 0, 0))],
        out_specs=[ospec] * 6,
        out_shape=[slab] * 6,
        compiler_params=pltpu.CompilerParams(
            dimension_semantics=("arbitrary",)),
    )(xt)
    scr, nx1, ny1, nx2, ny2, a2 = [s.reshape(B, NP) for s in slabs]

    mesh = plsc.VectorSubcoreMesh(core_axis_name="c", subcore_axis_name="s")
    sc_nms = functools.partial(
        pl.kernel,
        mesh=mesh,
        out_type=jax.ShapeDtypeStruct((B, _MAX_OBJ * _ROWL), jnp.float32),
        scratch_types=[pltpu.VMEM((NP,), jnp.float32)] * 6
        + [pltpu.VMEM((_MAX_OBJ * _ROWL,), jnp.float32)],
    )(functools.partial(_sc_nms_kernel, B=B, NP=NP))

    out = sc_nms(scr, nx1, ny1, nx2, ny2, a2)
    return out.reshape(B * _MAX_OBJ, _ROWL)[:, :7]


# speedup vs baseline: 1.0118x; 1.0118x over previous
import functools

import jax
import jax.numpy as jnp
from jax import lax
from jax.experimental import pallas as pl
from jax.experimental.pallas import tpu as pltpu
from jax.experimental.pallas import tpu_sc as plsc

_MAX_OBJ = 100
_IOU_THRES = 0.45
_SCORE_THRES = 0.25
_MAX_WH = 640.0
_NEG_INF = float("-inf")
_BIG_I32 = 2**30
_ROWL = 16


def _preproc_kernel(x_ref, scr_o, nx1_o, ny1_o, nx2_o, ny2_o, a2_o, *, NCLS):
    cx = x_ref[0, 0]
    cy = x_ref[0, 1]
    w = x_ref[0, 2]
    h = x_ref[0, 3]
    conf = x_ref[0, 4]

    cxr = cx.astype(jnp.bfloat16).astype(jnp.float32)
    cyr = cy.astype(jnp.bfloat16).astype(jnp.float32)
    wr = w.astype(jnp.bfloat16).astype(jnp.float32)
    hr = h.astype(jnp.bfloat16).astype(jnp.float32)
    bx1 = cxr - 0.5 * wr
    by1 = cyr - 0.5 * hr
    bx2 = cxr + 0.5 * wr
    by2 = cyr + 0.5 * hr

    sc0 = x_ref[0, 5] * conf

    def cls_body(c, carry):
        msc, cat = carry
        sc = x_ref[0, 5 + c] * conf
        cat = jnp.where(sc > msc, c, cat)
        msc = jnp.maximum(msc, sc)
        return msc, cat

    msc, cat = lax.fori_loop(1, NCLS, cls_body,
                             (sc0, jnp.zeros_like(sc0, jnp.int32)))
    catf = cat.astype(jnp.float32)

    nx1 = bx1 + catf * _MAX_WH
    ny1 = by1 + catf * _MAX_WH
    nx2 = bx2 + catf * _MAX_WH
    ny2 = by2 + catf * _MAX_WH

    scr_o[0] = jnp.where(msc > _SCORE_THRES, msc, _NEG_INF)
    nx1_o[0] = nx1
    ny1_o[0] = ny1
    nx2_o[0] = nx2
    ny2_o[0] = ny2
    a2_o[0] = (nx2 - nx1) * (ny2 - ny1)


def _xlane_reduce(v, op):
    lanes = lax.iota(jnp.int32, 16)
    for k in (1, 2, 4, 8):
        v = op(v, v.at[lanes ^ k].get(mode="promise_in_bounds"))
    return v


def _sc_nms_kernel(scr_h, nx1_h, ny1_h, nx2_h, ny2_h, a2_h, out_h,
                   scr_v, x1_v, y1_v, x2_v, y2_v, a2_v, rows_v,
                   *, B, NP):
    wid = lax.axis_index("s") * 2 + lax.axis_index("c")
    CH = NP // 16
    lanes = lax.iota(jnp.int32, 16)

    @pl.when(wid < B)
    def _run():
        b = wid
        bf = b.astype(jnp.float32)
        pltpu.sync_copy(scr_h.at[b], scr_v)
        pltpu.sync_copy(nx1_h.at[b], x1_v)
        pltpu.sync_copy(ny1_h.at[b], y1_v)
        pltpu.sync_copy(nx2_h.at[b], x2_v)
        pltpu.sync_copy(ny2_h.at[b], y2_v)
        pltpu.sync_copy(a2_h.at[b], a2_v)

        neg = jnp.full((16,), _NEG_INF, jnp.float32)
        big_i = jnp.full((16,), _BIG_I32, jnp.int32)

        def _amax_upd(news, c, vm, vi):
            better = (news > vm) | ((news == vm) & (c < vi))
            return jnp.where(better, news, vm), jnp.where(better, c, vi)

        def scan0_body(c, carry):
            vm, vi = carry
            s = scr_v[pl.ds(c * 16, 16)]
            return _amax_upd(s, c, vm, vi)

        vmax, vidxc = plsc.parallel_loop(
            0, CH, 1, unroll=4, carry=(neg, big_i))(scan0_body)

        def pick_body(i, carry):
            vm, vi = carry
            m = _xlane_reduce(vm, jnp.maximum)[0]
            ok = m > _NEG_INF
            cand = jnp.where(vm == m, vi * 16 + lanes, _BIG_I32)
            j = _xlane_reduce(cand, jnp.minimum)[0]

            cj16 = (j // 16) * 16
            jl = jnp.broadcast_to(j & 15, (16,))

            def splat_at(arr_ref):
                v = arr_ref[pl.ds(cj16, 16)]
                return v.at[jl].get(mode="promise_in_bounds")

            x1j = splat_at(x1_v)
            y1j = splat_at(y1_v)
            x2j = splat_at(x2_v)
            y2j = splat_at(y2_v)
            a1 = (x2j - x1j) * (y2j - y1j)

            catf = ((x1j + 0.5) / _MAX_WH).astype(jnp.int32) \
                .astype(jnp.float32)
            off = catf * _MAX_WH
            scorej = jnp.where(ok, m, 0.0)
            okf = jnp.where(ok, 1.0, 0.0)
            row = jnp.where(lanes == 0, bf,
                  jnp.where(lanes == 1, x1j - off,
                  jnp.where(lanes == 2, y1j - off,
                  jnp.where(lanes == 3, x2j - off,
                  jnp.where(lanes == 4, y2j - off,
                  jnp.where(lanes == 5, catf,
                  jnp.where(lanes == 6, scorej, 0.0))))))) * okf
            rows_v[pl.ds(i * _ROWL, 16)] = row

            sj = scr_v[pl.ds(cj16, 16)]
            scr_v[pl.ds(cj16, 16)] = jnp.where(lanes == (j & 15),
                                               _NEG_INF, sj)

            def chunk_body(c, carry2):
                vm2, vi2 = carry2
                ds = pl.ds(c * 16, 16)
                s = scr_v[ds]
                xx1 = jnp.maximum(x1j, x1_v[ds])
                yy1 = jnp.maximum(y1j, y1_v[ds])
                xx2 = jnp.minimum(x2j, x2_v[ds])
                yy2 = jnp.minimum(y2j, y2_v[ds])
                inter = (jnp.maximum(xx2 - xx1, 0.0)
                         * jnp.maximum(yy2 - yy1, 0.0))
                iou = inter / (a1 + a2_v[ds] - inter + 1e-9)
                news = jnp.where(iou > _IOU_THRES, neg, s)
                scr_v[ds] = news
                return _amax_upd(news, c, vm2, vi2)

            return plsc.parallel_loop(
                0, CH, 1, unroll=2, carry=(neg, big_i))(chunk_body)

        lax.fori_loop(0, _MAX_OBJ, pick_body, (vmax, vidxc))
        pltpu.sync_copy(rows_v, out_h.at[b])


@jax.jit
def kernel(x):
    B, N, C = x.shape
    R = (N + 127) // 128
    NP = R * 128
    NCLS = C - 5

    xp = jnp.pad(x, ((0, 0), (0, NP - N), (0, 0)))
    xt = xp.transpose(0, 2, 1).reshape(B, C, R, 128)

    slab = jax.ShapeDtypeStruct((B, R, 128), jnp.float32)
    ospec = pl.BlockSpec((1, R, 128), lambda b: (b, 0, 0))
    slabs = pl.pallas_call(
        functools.partial(_preproc_kernel, NCLS=NCLS),
        grid=(B,),
        in_specs=[pl.BlockSpec((1, C, R, 128), lambda b: (b, 0, 0, 0))],
        out_specs=[ospec] * 6,
        out_shape=[slab] * 6,
        compiler_params=pltpu.CompilerParams(
            dimension_semantics=("arbitrary",)),
    )(xt)
    scr, nx1, ny1, nx2, ny2, a2 = [s.reshape(B, NP) for s in slabs]

    mesh = plsc.VectorSubcoreMesh(core_axis_name="c", subcore_axis_name="s")
    sc_nms = functools.partial(
        pl.kernel,
        mesh=mesh,
        out_type=jax.ShapeDtypeStruct((B, _MAX_OBJ * _ROWL), jnp.float32),
        scratch_types=[pltpu.VMEM((NP,), jnp.float32)] * 6
        + [pltpu.VMEM((_MAX_OBJ * _ROWL,), jnp.float32)],
    )(functools.partial(_sc_nms_kernel, B=B, NP=NP))

    out = sc_nms(scr, nx1, ny1, nx2, ny2, a2)
    return out.reshape(B * _MAX_OBJ, _ROWL)[:, :7]
